# Initial kernel scaffold; baseline (speedup 1.0000x reference)
import jax
import jax.numpy as jnp
from jax.experimental import pallas as pl


def kernel(seq_len, table):
    raise NotImplementedError("write your pallas kernel here")


# speedup vs baseline: 2.7617x; 2.7617x over previous
import jax
import jax.numpy as jnp
from jax.experimental import pallas as pl


def _copy_block(in_ref, out_ref):
    out_ref[...] = in_ref[...]


def kernel(seq_len, table):
    del seq_len
    rows, dim = table.shape
    blk = 512
    out = pl.pallas_call(
        _copy_block,
        grid=(rows // blk,),
        in_specs=[pl.BlockSpec((blk, dim), lambda i: (i, 0))],
        out_specs=pl.BlockSpec((blk, dim), lambda i: (i, 0)),
        out_shape=jax.ShapeDtypeStruct((rows, dim), table.dtype),
    )(table)
    return out[None]
